# Initial kernel scaffold; baseline (speedup 1.0000x reference)
import jax
import jax.numpy as jnp
from jax.experimental import pallas as pl


def kernel(model_out, y, edge_index, train_idx):
    raise NotImplementedError("write your pallas kernel here")


# speedup vs baseline: 20.2554x; 20.2554x over previous
import functools

import jax
import jax.numpy as jnp
from jax import lax
from jax.experimental import pallas as pl
from jax.experimental.pallas import tpu as pltpu
from jax.experimental.pallas import tpu_sc as plsc

N = 50000
C = 16
E = 1600000
N_TRAIN = 25000
A1 = 0.9
A2 = 0.8
NPROP1 = 10
NPROP2 = 10

N_PAD = 51200
RPT = N_PAD // 16
CHUNK = 1024
NSTR = CHUNK // 128


def _cdiv(a, b):
    return (a + b - 1) // b


@functools.lru_cache(maxsize=None)
def _make_agg(cpt: int, gather: bool):
    mesh = plsc.VectorSubcoreMesh(
        core_axis_name="c", subcore_axis_name="s", num_cores=2, num_subcores=16
    )

    @functools.partial(
        pl.kernel,
        out_type=jax.ShapeDtypeStruct((2, N_PAD, C), jnp.float32),
        mesh=mesh,
        scratch_types=[
            pltpu.VMEM((16, 128), jnp.int32),
            pltpu.VMEM((CHUNK, C), jnp.float32),
            pltpu.VMEM((RPT // 8, C), jnp.float32),
            pltpu.VMEM_SHARED((N_PAD, C), jnp.float32),
            pltpu.SemaphoreType.DMA,
        ],
        compiler_params=pltpu.CompilerParams(use_tc_tiling_on_sc=False),
    )
    def agg(g_hbm, sd_hbm, out_hbm, sdv, rows, zbuf, acc, gsem):
        cid = lax.axis_index("c")
        sid = lax.axis_index("s")

        zrows = RPT // 8

        def _zb(i, carry):
            zbuf[i, :] = jnp.zeros((C,), jnp.float32)
            return carry

        lax.fori_loop(0, zrows, _zb, 0)

        if not gather:
            def _ob(i, carry):
                rows[i, :] = jnp.ones((C,), jnp.float32)
                return carry

            lax.fori_loop(0, CHUNK, _ob, 0)

        for k in range(8):
            pltpu.sync_copy(zbuf, acc.at[pl.ds(sid * RPT + k * zrows, zrows)])
        plsc.subcore_barrier()

        ch0 = cid * (cpt * 16) + sid * cpt

        def _chunk(i, carry):
            ch = ch0 + i
            pltpu.sync_copy(sd_hbm.at[ch], sdv)
            if gather:
                descs = [
                    pltpu.async_copy(
                        g_hbm.at[sdv.at[j]],
                        rows.at[pl.ds(j * 128, 128)],
                        gsem,
                    )
                    for j in range(NSTR)
                ]
                for d in descs:
                    d.wait()
            for j in range(NSTR):
                pltpu.sync_copy(
                    rows.at[pl.ds(j * 128, 128)],
                    acc.at[sdv.at[j + 8]],
                    add=True,
                )
            return carry

        lax.fori_loop(0, cpt, _chunk, 0)
        plsc.subcore_barrier()
        pltpu.sync_copy(
            acc.at[pl.ds(sid * RPT, RPT)],
            out_hbm.at[cid, pl.ds(sid * RPT, RPT)],
        )

    return agg


def _mk_sd(src, dst, total_chunks):
    pad = total_chunks * CHUNK - src.shape[0]
    s = jnp.concatenate([src, jnp.full((pad,), N, jnp.int32)])
    d = jnp.concatenate([dst, jnp.full((pad,), N, jnp.int32)])
    return jnp.concatenate(
        [s.reshape(total_chunks, 8, 128), d.reshape(total_chunks, 8, 128)],
        axis=1,
    )


_BLK = pl.BlockSpec((RPT, C), lambda i: (i, 0))
_BLK1 = pl.BlockSpec((RPT, 1), lambda i: (i, 0))
_BLKS = pl.BlockSpec((1, 1), lambda i: (0, 0))
_F32 = jnp.float32


def _prep_body(mo, y2, dS0, dS1, dD0, dD1, c0, c1,
               err0_o, g0_o, ds_o, dd_o, probs_o, cntm_o, sig_o):
    x = mo[...]
    m = jnp.max(x, axis=1, keepdims=True)
    lse = jnp.log(jnp.sum(jnp.exp(x - m), axis=1, keepdims=True)) + m
    probs = jnp.exp(x - lse)
    yoh = (jax.lax.broadcasted_iota(jnp.int32, x.shape, 1) == y2[...]).astype(_F32)
    cnt = c0[...] + c1[...]
    mask = (cnt > 0.0).astype(_F32)
    err0 = mask * (yoh - probs)
    degs = dS0[...] + dS1[...]
    degd = dD0[...] + dD1[...]
    dsm = jnp.where(degs > 0.0, jax.lax.rsqrt(jnp.maximum(degs, 1e-30)), 0.0)
    ddm = jnp.where(degd > 0.0, jax.lax.rsqrt(jnp.maximum(degd, 1e-30)), 0.0)
    err0_o[...] = err0
    g0_o[...] = dsm * err0
    ds_o[...] = dsm
    dd_o[...] = ddm
    probs_o[...] = probs
    cntm_o[...] = cnt

    @pl.when(pl.program_id(0) == 0)
    def _():
        sig_o[...] = jnp.zeros((1, 1), _F32)

    row = (pl.program_id(0) * RPT
           + jax.lax.broadcasted_iota(jnp.int32, x.shape, 0))
    contrib = jnp.sum(jnp.where(row < N, cnt * jnp.abs(err0), 0.0))
    sig_o[...] += jnp.full((1, 1), 1.0, _F32) * contrib


def _prep(mo, y2, degS, degD, cntp):
    shp = jax.ShapeDtypeStruct((N_PAD, C), _F32)
    return pl.pallas_call(
        _prep_body,
        grid=(16,),
        in_specs=[_BLK, _BLK1, _BLK, _BLK, _BLK, _BLK, _BLK, _BLK],
        out_specs=[_BLK, _BLK, _BLK, _BLK, _BLK, _BLK, _BLKS],
        out_shape=[shp, shp, shp, shp, shp, shp,
                   jax.ShapeDtypeStruct((1, 1), _F32)],
    )(mo, y2, degS[0], degS[1], degD[0], degD[1], cntp[0], cntp[1])


def _step_body(alpha, clip, p0, p1, h0, ddm, dsm, h_o, g_o):
    v = alpha * ddm[...] * (p0[...] + p1[...]) + (1.0 - alpha) * h0[...]
    if clip:
        v = jnp.clip(v, 0.0, 1.0)
    h_o[...] = v
    g_o[...] = dsm[...] * v


def _step(alpha, clip, p, h0, ddm, dsm):
    shp = jax.ShapeDtypeStruct((N_PAD, C), _F32)
    return pl.pallas_call(
        functools.partial(_step_body, alpha, clip),
        grid=(16,),
        in_specs=[_BLK, _BLK, _BLK, _BLK, _BLK],
        out_specs=[_BLK, _BLK],
        out_shape=[shp, shp],
    )(p[0], p[1], h0, ddm, dsm)


def _mid_body(err, probs, cntm, y2, dsm, sig, h0_o, g_o):
    sigma = sig[0, 0] / float(N_TRAIN)
    e = err[...]
    l1 = jnp.sum(jnp.abs(e), axis=1, keepdims=True)
    scale = jnp.clip(sigma / (l1 + 1e-9), 0.0, 1000.0)
    out = probs[...] + scale * e
    yoh = (jax.lax.broadcasted_iota(jnp.int32, e.shape, 1) == y2[...]).astype(_F32)
    h0 = jnp.where(cntm[...] > 0.0, yoh, out)
    h0_o[...] = h0
    g_o[...] = dsm[...] * h0


def _mid(err, probs, cntm, y2, dsm, sig):
    shp = jax.ShapeDtypeStruct((N_PAD, C), _F32)
    return pl.pallas_call(
        _mid_body,
        grid=(16,),
        in_specs=[_BLK, _BLK, _BLK, _BLK1, _BLK, _BLKS],
        out_specs=[_BLK, _BLK],
        out_shape=[shp, shp],
    )(err, probs, cntm, y2, dsm, sig)


def kernel(model_out, y, edge_index, train_idx):
    src = edge_index[0].astype(jnp.int32)
    dst = edge_index[1].astype(jnp.int32)
    train_idx = train_idx.astype(jnp.int32)

    cpt_e = _cdiv(E, 32 * CHUNK)
    cpt_t = _cdiv(N_TRAIN, 32 * CHUNK)

    sd_edges = _mk_sd(src, dst, cpt_e * 32)
    sd_src = _mk_sd(src, src, cpt_e * 32)
    sd_dst = _mk_sd(dst, dst, cpt_e * 32)
    sd_train = _mk_sd(train_idx, train_idx, cpt_t * 32)

    mo = jnp.pad(model_out, ((0, N_PAD - N), (0, 0)))
    y2 = jnp.pad(y, (0, N_PAD - N)).reshape(N_PAD, 1)

    agg = _make_agg(cpt_e, True)
    ones_big = _make_agg(cpt_e, False)
    ones_small = _make_agg(cpt_t, False)

    dummy_g = mo
    degS = ones_big(dummy_g, sd_src)
    degD = ones_big(dummy_g, sd_dst)
    cntp = ones_small(dummy_g, sd_train)

    err0, g, dsm, ddm, probs, cntm, sig = _prep(mo, y2, degS, degD, cntp)

    err = err0
    for _ in range(NPROP1):
        p = agg(g, sd_edges)
        err, g = _step(A1, False, p, err0, ddm, dsm)

    h0, g = _mid(err, probs, cntm, y2, dsm, sig)

    h = h0
    for _ in range(NPROP2):
        p = agg(g, sd_edges)
        h, g = _step(A2, True, p, h0, ddm, dsm)

    return h[:N]


# speedup vs baseline: 20.6580x; 1.0199x over previous
import functools

import jax
import jax.numpy as jnp
from jax import lax
from jax.experimental import pallas as pl
from jax.experimental.pallas import tpu as pltpu
from jax.experimental.pallas import tpu_sc as plsc

N = 50000
C = 16
E = 1600000
N_TRAIN = 25000
A1 = 0.9
A2 = 0.8
NPROP1 = 10
NPROP2 = 10

N_PAD = 51200
RPT = N_PAD // 16
CHUNK = 1024
NSTR = CHUNK // 128


def _cdiv(a, b):
    return (a + b - 1) // b


@functools.lru_cache(maxsize=None)
def _make_agg(cpt: int, gather: bool):
    mesh = plsc.VectorSubcoreMesh(
        core_axis_name="c", subcore_axis_name="s", num_cores=2, num_subcores=16
    )

    @functools.partial(
        pl.kernel,
        out_type=jax.ShapeDtypeStruct((2, N_PAD, C), jnp.float32),
        mesh=mesh,
        scratch_types=[
            pltpu.VMEM((2, CHUNK), jnp.int32),
            pltpu.VMEM((CHUNK, C), jnp.float32),
            pltpu.VMEM((RPT // 8, C), jnp.float32),
            pltpu.VMEM_SHARED((N_PAD, C), jnp.float32),
            pltpu.SemaphoreType.DMA,
        ],
        compiler_params=pltpu.CompilerParams(use_tc_tiling_on_sc=False),
    )
    def agg(g_hbm, sd_hbm, out_hbm, sdv, rows, zbuf, acc, gsem):
        cid = lax.axis_index("c")
        sid = lax.axis_index("s")

        zrows = RPT // 8

        def _zb(i, carry):
            zbuf[i, :] = jnp.zeros((C,), jnp.float32)
            return carry

        lax.fori_loop(0, zrows, _zb, 0)

        if not gather:
            def _ob(i, carry):
                rows[i, :] = jnp.ones((C,), jnp.float32)
                return carry

            lax.fori_loop(0, CHUNK, _ob, 0)

        for k in range(8):
            pltpu.sync_copy(zbuf, acc.at[pl.ds(sid * RPT + k * zrows, zrows)])
        plsc.subcore_barrier()

        ch0 = cid * (cpt * 16) + sid * cpt

        def _chunk(i, carry):
            ch = ch0 + i
            pltpu.sync_copy(sd_hbm.at[ch], sdv)
            if gather:
                pltpu.async_copy(g_hbm.at[sdv.at[0]], rows, gsem).wait()
            pltpu.sync_copy(rows, acc.at[sdv.at[1]], add=True)
            return carry

        lax.fori_loop(0, cpt, _chunk, 0)
        plsc.subcore_barrier()
        pltpu.sync_copy(
            acc.at[pl.ds(sid * RPT, RPT)],
            out_hbm.at[cid, pl.ds(sid * RPT, RPT)],
        )

    return agg


def _mk_sd(src, dst, total_chunks):
    pad = total_chunks * CHUNK - src.shape[0]
    s = jnp.concatenate([src, jnp.full((pad,), N, jnp.int32)])
    d = jnp.concatenate([dst, jnp.full((pad,), N, jnp.int32)])
    return jnp.stack(
        [s.reshape(total_chunks, CHUNK), d.reshape(total_chunks, CHUNK)],
        axis=1,
    )


_BLK = pl.BlockSpec((RPT, C), lambda i: (i, 0))
_BLK1 = pl.BlockSpec((RPT, 1), lambda i: (i, 0))
_BLKS = pl.BlockSpec((1, 1), lambda i: (0, 0))
_F32 = jnp.float32


def _prep_body(mo, y2, dS0, dS1, dD0, dD1, c0, c1,
               err0_o, g0_o, ds_o, dd_o, probs_o, cntm_o, sig_o):
    x = mo[...]
    m = jnp.max(x, axis=1, keepdims=True)
    lse = jnp.log(jnp.sum(jnp.exp(x - m), axis=1, keepdims=True)) + m
    probs = jnp.exp(x - lse)
    yoh = (jax.lax.broadcasted_iota(jnp.int32, x.shape, 1) == y2[...]).astype(_F32)
    cnt = c0[...] + c1[...]
    mask = (cnt > 0.0).astype(_F32)
    err0 = mask * (yoh - probs)
    degs = dS0[...] + dS1[...]
    degd = dD0[...] + dD1[...]
    dsm = jnp.where(degs > 0.0, jax.lax.rsqrt(jnp.maximum(degs, 1e-30)), 0.0)
    ddm = jnp.where(degd > 0.0, jax.lax.rsqrt(jnp.maximum(degd, 1e-30)), 0.0)
    err0_o[...] = err0
    g0_o[...] = dsm * err0
    ds_o[...] = dsm
    dd_o[...] = ddm
    probs_o[...] = probs
    cntm_o[...] = cnt

    @pl.when(pl.program_id(0) == 0)
    def _():
        sig_o[...] = jnp.zeros((1, 1), _F32)

    row = (pl.program_id(0) * RPT
           + jax.lax.broadcasted_iota(jnp.int32, x.shape, 0))
    contrib = jnp.sum(jnp.where(row < N, cnt * jnp.abs(err0), 0.0))
    sig_o[...] += jnp.full((1, 1), 1.0, _F32) * contrib


def _prep(mo, y2, degS, degD, cntp):
    shp = jax.ShapeDtypeStruct((N_PAD, C), _F32)
    return pl.pallas_call(
        _prep_body,
        grid=(16,),
        in_specs=[_BLK, _BLK1, _BLK, _BLK, _BLK, _BLK, _BLK, _BLK],
        out_specs=[_BLK, _BLK, _BLK, _BLK, _BLK, _BLK, _BLKS],
        out_shape=[shp, shp, shp, shp, shp, shp,
                   jax.ShapeDtypeStruct((1, 1), _F32)],
    )(mo, y2, degS[0], degS[1], degD[0], degD[1], cntp[0], cntp[1])


def _step_body(alpha, clip, p0, p1, h0, ddm, dsm, h_o, g_o):
    v = alpha * ddm[...] * (p0[...] + p1[...]) + (1.0 - alpha) * h0[...]
    if clip:
        v = jnp.clip(v, 0.0, 1.0)
    h_o[...] = v
    g_o[...] = dsm[...] * v


def _step(alpha, clip, p, h0, ddm, dsm):
    shp = jax.ShapeDtypeStruct((N_PAD, C), _F32)
    return pl.pallas_call(
        functools.partial(_step_body, alpha, clip),
        grid=(16,),
        in_specs=[_BLK, _BLK, _BLK, _BLK, _BLK],
        out_specs=[_BLK, _BLK],
        out_shape=[shp, shp],
    )(p[0], p[1], h0, ddm, dsm)


def _mid_body(err, probs, cntm, y2, dsm, sig, h0_o, g_o):
    sigma = sig[0, 0] / float(N_TRAIN)
    e = err[...]
    l1 = jnp.sum(jnp.abs(e), axis=1, keepdims=True)
    scale = jnp.clip(sigma / (l1 + 1e-9), 0.0, 1000.0)
    out = probs[...] + scale * e
    yoh = (jax.lax.broadcasted_iota(jnp.int32, e.shape, 1) == y2[...]).astype(_F32)
    h0 = jnp.where(cntm[...] > 0.0, yoh, out)
    h0_o[...] = h0
    g_o[...] = dsm[...] * h0


def _mid(err, probs, cntm, y2, dsm, sig):
    shp = jax.ShapeDtypeStruct((N_PAD, C), _F32)
    return pl.pallas_call(
        _mid_body,
        grid=(16,),
        in_specs=[_BLK, _BLK, _BLK, _BLK1, _BLK, _BLKS],
        out_specs=[_BLK, _BLK],
        out_shape=[shp, shp],
    )(err, probs, cntm, y2, dsm, sig)


def kernel(model_out, y, edge_index, train_idx):
    src = edge_index[0].astype(jnp.int32)
    dst = edge_index[1].astype(jnp.int32)
    train_idx = train_idx.astype(jnp.int32)

    cpt_e = _cdiv(E, 32 * CHUNK)
    cpt_t = _cdiv(N_TRAIN, 32 * CHUNK)

    sd_edges = _mk_sd(src, dst, cpt_e * 32)
    sd_src = _mk_sd(src, src, cpt_e * 32)
    sd_dst = _mk_sd(dst, dst, cpt_e * 32)
    sd_train = _mk_sd(train_idx, train_idx, cpt_t * 32)

    mo = jnp.pad(model_out, ((0, N_PAD - N), (0, 0)))
    y2 = jnp.pad(y, (0, N_PAD - N)).reshape(N_PAD, 1)

    agg = _make_agg(cpt_e, True)
    ones_big = _make_agg(cpt_e, False)
    ones_small = _make_agg(cpt_t, False)

    dummy_g = mo
    degS = ones_big(dummy_g, sd_src)
    degD = ones_big(dummy_g, sd_dst)
    cntp = ones_small(dummy_g, sd_train)

    err0, g, dsm, ddm, probs, cntm, sig = _prep(mo, y2, degS, degD, cntp)

    err = err0
    for _ in range(NPROP1):
        p = agg(g, sd_edges)
        err, g = _step(A1, False, p, err0, ddm, dsm)

    h0, g = _mid(err, probs, cntm, y2, dsm, sig)

    h = h0
    for _ in range(NPROP2):
        p = agg(g, sd_edges)
        h, g = _step(A2, True, p, h0, ddm, dsm)

    return h[:N]


# speedup vs baseline: 25.4818x; 1.2335x over previous
import functools

import jax
import jax.numpy as jnp
from jax import lax
from jax.experimental import pallas as pl
from jax.experimental.pallas import tpu as pltpu
from jax.experimental.pallas import tpu_sc as plsc

N = 50000
C = 16
E = 1600000
N_TRAIN = 25000
A1 = 0.9
A2 = 0.8
NPROP1 = 10
NPROP2 = 10

N_PAD = 51200
RPT = N_PAD // 16
CHUNK = 1024
NSTR = CHUNK // 128


def _cdiv(a, b):
    return (a + b - 1) // b


@functools.lru_cache(maxsize=None)
def _make_agg(cpt: int, gather: bool):
    mesh = plsc.VectorSubcoreMesh(
        core_axis_name="c", subcore_axis_name="s", num_cores=2, num_subcores=16
    )

    @functools.partial(
        pl.kernel,
        out_type=jax.ShapeDtypeStruct((2, N_PAD, C), jnp.float32),
        mesh=mesh,
        scratch_types=[
            pltpu.VMEM((2, 2, CHUNK), jnp.int32),
            pltpu.VMEM((2, CHUNK, C), jnp.float32),
            pltpu.VMEM((RPT // 8, C), jnp.float32),
            pltpu.VMEM_SHARED((N_PAD, C), jnp.float32),
            pltpu.SemaphoreType.DMA,
            pltpu.SemaphoreType.DMA,
            pltpu.SemaphoreType.DMA,
            pltpu.SemaphoreType.DMA,
        ],
        compiler_params=pltpu.CompilerParams(use_tc_tiling_on_sc=False),
    )
    def agg(g_hbm, sd_hbm, out_hbm, sdv, rows, zbuf, acc,
            gsem0, gsem1, ssem0, ssem1):
        cid = lax.axis_index("c")
        sid = lax.axis_index("s")

        zrows = RPT // 8

        def _zb(i, carry):
            zbuf[i, :] = jnp.zeros((C,), jnp.float32)
            return carry

        lax.fori_loop(0, zrows, _zb, 0)

        if not gather:
            def _ob(i, carry):
                rows[0, i, :] = jnp.ones((C,), jnp.float32)
                return carry

            lax.fori_loop(0, CHUNK, _ob, 0)

        for k in range(8):
            pltpu.sync_copy(zbuf, acc.at[pl.ds(sid * RPT + k * zrows, zrows)])
        plsc.subcore_barrier()

        ch0 = cid * (cpt * 16) + sid * cpt

        if not gather:
            def _chunk(i, carry):
                pltpu.sync_copy(sd_hbm.at[ch0 + i], sdv.at[0])
                pltpu.sync_copy(rows.at[0], acc.at[sdv.at[0, 1]], add=True)
                return carry

            lax.fori_loop(0, cpt, _chunk, 0)
        else:
            gsem = (gsem0, gsem1)
            ssem = (ssem0, ssem1)

            def load_fire(ch, b):
                pltpu.sync_copy(sd_hbm.at[ch], sdv.at[b])
                pltpu.async_copy(
                    g_hbm.at[sdv.at[b, 0]], rows.at[b], gsem[b]
                )

            def wait_g(b):
                pltpu.make_async_copy(
                    g_hbm.at[sdv.at[b, 0]], rows.at[b], gsem[b]
                ).wait()

            def fire_s(b):
                pltpu.async_copy(
                    rows.at[b], acc.at[sdv.at[b, 1]], ssem[b], add=True
                )

            def wait_s(b):
                pltpu.make_async_copy(
                    rows.at[b], acc.at[sdv.at[b, 1]], ssem[b]
                ).wait()

            assert cpt >= 3 and (cpt - 3) % 2 == 0
            load_fire(ch0 + 0, 0)
            load_fire(ch0 + 1, 1)
            wait_g(0)
            fire_s(0)
            wait_s(0)
            load_fire(ch0 + 2, 0)
            wait_g(1)
            fire_s(1)

            def _pair(k, carry):
                ch = ch0 + 3 + 2 * k
                wait_s(1)
                load_fire(ch, 1)
                wait_g(0)
                fire_s(0)
                wait_s(0)
                load_fire(ch + 1, 0)
                wait_g(1)
                fire_s(1)
                return carry

            lax.fori_loop(0, (cpt - 3) // 2, _pair, 0)
            wait_g(0)
            fire_s(0)
            wait_s(1)
            wait_s(0)
        plsc.subcore_barrier()
        pltpu.sync_copy(
            acc.at[pl.ds(sid * RPT, RPT)],
            out_hbm.at[cid, pl.ds(sid * RPT, RPT)],
        )

    return agg


def _mk_sd(src, dst, total_chunks):
    pad = total_chunks * CHUNK - src.shape[0]
    s = jnp.concatenate([src, jnp.full((pad,), N, jnp.int32)])
    d = jnp.concatenate([dst, jnp.full((pad,), N, jnp.int32)])
    return jnp.stack(
        [s.reshape(total_chunks, CHUNK), d.reshape(total_chunks, CHUNK)],
        axis=1,
    )


_BLK = pl.BlockSpec((RPT, C), lambda i: (i, 0))
_BLK1 = pl.BlockSpec((RPT, 1), lambda i: (i, 0))
_BLKS = pl.BlockSpec((1, 1), lambda i: (0, 0))
_F32 = jnp.float32


def _prep_body(mo, y2, dS0, dS1, dD0, dD1, c0, c1,
               err0_o, g0_o, ds_o, dd_o, probs_o, cntm_o, sig_o):
    x = mo[...]
    m = jnp.max(x, axis=1, keepdims=True)
    lse = jnp.log(jnp.sum(jnp.exp(x - m), axis=1, keepdims=True)) + m
    probs = jnp.exp(x - lse)
    yoh = (jax.lax.broadcasted_iota(jnp.int32, x.shape, 1) == y2[...]).astype(_F32)
    cnt = c0[...] + c1[...]
    mask = (cnt > 0.0).astype(_F32)
    err0 = mask * (yoh - probs)
    degs = dS0[...] + dS1[...]
    degd = dD0[...] + dD1[...]
    dsm = jnp.where(degs > 0.0, jax.lax.rsqrt(jnp.maximum(degs, 1e-30)), 0.0)
    ddm = jnp.where(degd > 0.0, jax.lax.rsqrt(jnp.maximum(degd, 1e-30)), 0.0)
    err0_o[...] = err0
    g0_o[...] = dsm * err0
    ds_o[...] = dsm
    dd_o[...] = ddm
    probs_o[...] = probs
    cntm_o[...] = cnt

    @pl.when(pl.program_id(0) == 0)
    def _():
        sig_o[...] = jnp.zeros((1, 1), _F32)

    row = (pl.program_id(0) * RPT
           + jax.lax.broadcasted_iota(jnp.int32, x.shape, 0))
    contrib = jnp.sum(jnp.where(row < N, cnt * jnp.abs(err0), 0.0))
    sig_o[...] += jnp.full((1, 1), 1.0, _F32) * contrib


def _prep(mo, y2, degS, degD, cntp):
    shp = jax.ShapeDtypeStruct((N_PAD, C), _F32)
    return pl.pallas_call(
        _prep_body,
        grid=(16,),
        in_specs=[_BLK, _BLK1, _BLK, _BLK, _BLK, _BLK, _BLK, _BLK],
        out_specs=[_BLK, _BLK, _BLK, _BLK, _BLK, _BLK, _BLKS],
        out_shape=[shp, shp, shp, shp, shp, shp,
                   jax.ShapeDtypeStruct((1, 1), _F32)],
    )(mo, y2, degS[0], degS[1], degD[0], degD[1], cntp[0], cntp[1])


def _step_body(alpha, clip, p0, p1, h0, ddm, dsm, h_o, g_o):
    v = alpha * ddm[...] * (p0[...] + p1[...]) + (1.0 - alpha) * h0[...]
    if clip:
        v = jnp.clip(v, 0.0, 1.0)
    h_o[...] = v
    g_o[...] = dsm[...] * v


def _step(alpha, clip, p, h0, ddm, dsm):
    shp = jax.ShapeDtypeStruct((N_PAD, C), _F32)
    return pl.pallas_call(
        functools.partial(_step_body, alpha, clip),
        grid=(16,),
        in_specs=[_BLK, _BLK, _BLK, _BLK, _BLK],
        out_specs=[_BLK, _BLK],
        out_shape=[shp, shp],
    )(p[0], p[1], h0, ddm, dsm)


def _mid_body(err, probs, cntm, y2, dsm, sig, h0_o, g_o):
    sigma = sig[0, 0] / float(N_TRAIN)
    e = err[...]
    l1 = jnp.sum(jnp.abs(e), axis=1, keepdims=True)
    scale = jnp.clip(sigma / (l1 + 1e-9), 0.0, 1000.0)
    out = probs[...] + scale * e
    yoh = (jax.lax.broadcasted_iota(jnp.int32, e.shape, 1) == y2[...]).astype(_F32)
    h0 = jnp.where(cntm[...] > 0.0, yoh, out)
    h0_o[...] = h0
    g_o[...] = dsm[...] * h0


def _mid(err, probs, cntm, y2, dsm, sig):
    shp = jax.ShapeDtypeStruct((N_PAD, C), _F32)
    return pl.pallas_call(
        _mid_body,
        grid=(16,),
        in_specs=[_BLK, _BLK, _BLK, _BLK1, _BLK, _BLKS],
        out_specs=[_BLK, _BLK],
        out_shape=[shp, shp],
    )(err, probs, cntm, y2, dsm, sig)


def kernel(model_out, y, edge_index, train_idx):
    src = edge_index[0].astype(jnp.int32)
    dst = edge_index[1].astype(jnp.int32)
    train_idx = train_idx.astype(jnp.int32)

    cpt_e = _cdiv(E, 32 * CHUNK)
    cpt_t = _cdiv(N_TRAIN, 32 * CHUNK)

    sd_edges = _mk_sd(src, dst, cpt_e * 32)
    sd_src = _mk_sd(src, src, cpt_e * 32)
    sd_dst = _mk_sd(dst, dst, cpt_e * 32)
    sd_train = _mk_sd(train_idx, train_idx, cpt_t * 32)

    mo = jnp.pad(model_out, ((0, N_PAD - N), (0, 0)))
    y2 = jnp.pad(y, (0, N_PAD - N)).reshape(N_PAD, 1)

    agg = _make_agg(cpt_e, True)
    ones_big = _make_agg(cpt_e, False)
    ones_small = _make_agg(cpt_t, False)

    dummy_g = mo
    degS = ones_big(dummy_g, sd_src)
    degD = ones_big(dummy_g, sd_dst)
    cntp = ones_small(dummy_g, sd_train)

    err0, g, dsm, ddm, probs, cntm, sig = _prep(mo, y2, degS, degD, cntp)

    err = err0
    for _ in range(NPROP1):
        p = agg(g, sd_edges)
        err, g = _step(A1, False, p, err0, ddm, dsm)

    h0, g = _mid(err, probs, cntm, y2, dsm, sig)

    h = h0
    for _ in range(NPROP2):
        p = agg(g, sd_edges)
        h, g = _step(A2, True, p, h0, ddm, dsm)

    return h[:N]


# speedup vs baseline: 26.4695x; 1.0388x over previous
import functools

import jax
import jax.numpy as jnp
from jax import lax
from jax.experimental import pallas as pl
from jax.experimental.pallas import tpu as pltpu
from jax.experimental.pallas import tpu_sc as plsc

N = 50000
C = 16
E = 1600000
N_TRAIN = 25000
A1 = 0.9
A2 = 0.8
NPROP1 = 10
NPROP2 = 10

N_PAD = 51200
RPT = N_PAD // 16
CHUNK = 784
NBUF = 4


def _cdiv(a, b):
    return (a + b - 1) // b


@functools.lru_cache(maxsize=None)
def _make_agg(cpt: int, gather: bool):
    mesh = plsc.VectorSubcoreMesh(
        core_axis_name="c", subcore_axis_name="s", num_cores=2, num_subcores=16
    )

    @functools.partial(
        pl.kernel,
        out_type=jax.ShapeDtypeStruct((2, N_PAD, C), jnp.float32),
        mesh=mesh,
        scratch_types=[
            pltpu.VMEM((NBUF, 2, CHUNK), jnp.int32),
            pltpu.VMEM((NBUF, CHUNK, C), jnp.float32),
            pltpu.VMEM((RPT // 8, C), jnp.float32),
            pltpu.VMEM_SHARED((N_PAD, C), jnp.float32),
        ] + [pltpu.SemaphoreType.DMA] * (2 * NBUF),
        compiler_params=pltpu.CompilerParams(use_tc_tiling_on_sc=False),
    )
    def agg(g_hbm, sd_hbm, out_hbm, sdv, rows, zbuf, acc, *sems):
        gsem = sems[:NBUF]
        ssem = sems[NBUF:]
        cid = lax.axis_index("c")
        sid = lax.axis_index("s")

        zrows = RPT // 8

        def _zb(i, carry):
            zbuf[i, :] = jnp.zeros((C,), jnp.float32)
            return carry

        lax.fori_loop(0, zrows, _zb, 0)

        if not gather:
            def _ob(i, carry):
                rows[0, i, :] = jnp.ones((C,), jnp.float32)
                return carry

            lax.fori_loop(0, CHUNK, _ob, 0)

        for k in range(8):
            pltpu.sync_copy(zbuf, acc.at[pl.ds(sid * RPT + k * zrows, zrows)])
        plsc.subcore_barrier()

        ch0 = cid * (cpt * 16) + sid * cpt

        if not gather:
            def _chunk(i, carry):
                pltpu.sync_copy(sd_hbm.at[ch0 + i], sdv.at[0])
                pltpu.sync_copy(rows.at[0], acc.at[sdv.at[0, 1]], add=True)
                return carry

            lax.fori_loop(0, cpt, _chunk, 0)
        else:
            def load_fire(ch, b):
                pltpu.sync_copy(sd_hbm.at[ch], sdv.at[b])
                pltpu.async_copy(
                    g_hbm.at[sdv.at[b, 0]], rows.at[b], gsem[b]
                )

            def wait_g(b):
                pltpu.make_async_copy(
                    g_hbm.at[sdv.at[b, 0]], rows.at[b], gsem[b]
                ).wait()

            def fire_s(b):
                pltpu.async_copy(
                    rows.at[b], acc.at[sdv.at[b, 1]], ssem[b], add=True
                )

            def wait_s(b):
                pltpu.make_async_copy(
                    rows.at[b], acc.at[sdv.at[b, 1]], ssem[b]
                ).wait()

            assert cpt >= NBUF and (cpt - NBUF) % NBUF == 0
            load_fire(ch0 + 0, 0)
            load_fire(ch0 + 1, 1)
            wait_g(0)
            fire_s(0)
            load_fire(ch0 + 2, 2)
            wait_g(1)
            fire_s(1)
            load_fire(ch0 + 3, 3)

            def _quad(k, carry):
                ch = ch0 + NBUF + NBUF * k
                for j in range(NBUF):
                    wait_g((j + 2) % NBUF)
                    fire_s((j + 2) % NBUF)
                    wait_s(j)
                    load_fire(ch + j, j)
                return carry

            lax.fori_loop(0, (cpt - NBUF) // NBUF, _quad, 0)
            wait_g(2)
            fire_s(2)
            wait_g(3)
            fire_s(3)
            for j in range(NBUF):
                wait_s(j)
        plsc.subcore_barrier()
        pltpu.sync_copy(
            acc.at[pl.ds(sid * RPT, RPT)],
            out_hbm.at[cid, pl.ds(sid * RPT, RPT)],
        )

    return agg


def _mk_sd(src, dst, total_chunks):
    pad = total_chunks * CHUNK - src.shape[0]
    s = jnp.concatenate([src, jnp.full((pad,), N, jnp.int32)])
    d = jnp.concatenate([dst, jnp.full((pad,), N, jnp.int32)])
    return jnp.stack(
        [s.reshape(total_chunks, CHUNK), d.reshape(total_chunks, CHUNK)],
        axis=1,
    )


_BLK = pl.BlockSpec((RPT, C), lambda i: (i, 0))
_BLK1 = pl.BlockSpec((RPT, 1), lambda i: (i, 0))
_BLKS = pl.BlockSpec((1, 1), lambda i: (0, 0))
_F32 = jnp.float32


def _prep_body(mo, y2, dS0, dS1, dD0, dD1, c0, c1,
               err0_o, g0_o, ds_o, dd_o, probs_o, cntm_o, sig_o):
    x = mo[...]
    m = jnp.max(x, axis=1, keepdims=True)
    lse = jnp.log(jnp.sum(jnp.exp(x - m), axis=1, keepdims=True)) + m
    probs = jnp.exp(x - lse)
    yoh = (jax.lax.broadcasted_iota(jnp.int32, x.shape, 1) == y2[...]).astype(_F32)
    cnt = c0[...] + c1[...]
    mask = (cnt > 0.0).astype(_F32)
    err0 = mask * (yoh - probs)
    degs = dS0[...] + dS1[...]
    degd = dD0[...] + dD1[...]
    dsm = jnp.where(degs > 0.0, jax.lax.rsqrt(jnp.maximum(degs, 1e-30)), 0.0)
    ddm = jnp.where(degd > 0.0, jax.lax.rsqrt(jnp.maximum(degd, 1e-30)), 0.0)
    err0_o[...] = err0
    g0_o[...] = dsm * err0
    ds_o[...] = dsm
    dd_o[...] = ddm
    probs_o[...] = probs
    cntm_o[...] = cnt

    @pl.when(pl.program_id(0) == 0)
    def _():
        sig_o[...] = jnp.zeros((1, 1), _F32)

    row = (pl.program_id(0) * RPT
           + jax.lax.broadcasted_iota(jnp.int32, x.shape, 0))
    contrib = jnp.sum(jnp.where(row < N, cnt * jnp.abs(err0), 0.0))
    sig_o[...] += jnp.full((1, 1), 1.0, _F32) * contrib


def _prep(mo, y2, degS, degD, cntp):
    shp = jax.ShapeDtypeStruct((N_PAD, C), _F32)
    return pl.pallas_call(
        _prep_body,
        grid=(16,),
        in_specs=[_BLK, _BLK1, _BLK, _BLK, _BLK, _BLK, _BLK, _BLK],
        out_specs=[_BLK, _BLK, _BLK, _BLK, _BLK, _BLK, _BLKS],
        out_shape=[shp, shp, shp, shp, shp, shp,
                   jax.ShapeDtypeStruct((1, 1), _F32)],
    )(mo, y2, degS[0], degS[1], degD[0], degD[1], cntp[0], cntp[1])


def _step_body(alpha, clip, p0, p1, h0, ddm, dsm, h_o, g_o):
    v = alpha * ddm[...] * (p0[...] + p1[...]) + (1.0 - alpha) * h0[...]
    if clip:
        v = jnp.clip(v, 0.0, 1.0)
    h_o[...] = v
    g_o[...] = dsm[...] * v


def _step(alpha, clip, p, h0, ddm, dsm):
    shp = jax.ShapeDtypeStruct((N_PAD, C), _F32)
    return pl.pallas_call(
        functools.partial(_step_body, alpha, clip),
        grid=(16,),
        in_specs=[_BLK, _BLK, _BLK, _BLK, _BLK],
        out_specs=[_BLK, _BLK],
        out_shape=[shp, shp],
    )(p[0], p[1], h0, ddm, dsm)


def _mid_body(err, probs, cntm, y2, dsm, sig, h0_o, g_o):
    sigma = sig[0, 0] / float(N_TRAIN)
    e = err[...]
    l1 = jnp.sum(jnp.abs(e), axis=1, keepdims=True)
    scale = jnp.clip(sigma / (l1 + 1e-9), 0.0, 1000.0)
    out = probs[...] + scale * e
    yoh = (jax.lax.broadcasted_iota(jnp.int32, e.shape, 1) == y2[...]).astype(_F32)
    h0 = jnp.where(cntm[...] > 0.0, yoh, out)
    h0_o[...] = h0
    g_o[...] = dsm[...] * h0


def _mid(err, probs, cntm, y2, dsm, sig):
    shp = jax.ShapeDtypeStruct((N_PAD, C), _F32)
    return pl.pallas_call(
        _mid_body,
        grid=(16,),
        in_specs=[_BLK, _BLK, _BLK, _BLK1, _BLK, _BLKS],
        out_specs=[_BLK, _BLK],
        out_shape=[shp, shp],
    )(err, probs, cntm, y2, dsm, sig)


def kernel(model_out, y, edge_index, train_idx):
    src = edge_index[0].astype(jnp.int32)
    dst = edge_index[1].astype(jnp.int32)
    train_idx = train_idx.astype(jnp.int32)

    cpt_e = _cdiv(E, 32 * CHUNK)
    cpt_t = _cdiv(N_TRAIN, 32 * CHUNK)

    sd_edges = _mk_sd(src, dst, cpt_e * 32)
    sd_src = _mk_sd(src, src, cpt_e * 32)
    sd_dst = _mk_sd(dst, dst, cpt_e * 32)
    sd_train = _mk_sd(train_idx, train_idx, cpt_t * 32)

    mo = jnp.pad(model_out, ((0, N_PAD - N), (0, 0)))
    y2 = jnp.pad(y, (0, N_PAD - N)).reshape(N_PAD, 1)

    agg = _make_agg(cpt_e, True)
    ones_big = _make_agg(cpt_e, False)
    ones_small = _make_agg(cpt_t, False)

    dummy_g = mo
    degS = ones_big(dummy_g, sd_src)
    degD = ones_big(dummy_g, sd_dst)
    cntp = ones_small(dummy_g, sd_train)

    err0, g, dsm, ddm, probs, cntm, sig = _prep(mo, y2, degS, degD, cntp)

    err = err0
    for _ in range(NPROP1):
        p = agg(g, sd_edges)
        err, g = _step(A1, False, p, err0, ddm, dsm)

    h0, g = _mid(err, probs, cntm, y2, dsm, sig)

    h = h0
    for _ in range(NPROP2):
        p = agg(g, sd_edges)
        h, g = _step(A2, True, p, h0, ddm, dsm)

    return h[:N]


# speedup vs baseline: 26.6772x; 1.0079x over previous
import functools

import jax
import jax.numpy as jnp
from jax import lax
from jax.experimental import pallas as pl
from jax.experimental.pallas import tpu as pltpu
from jax.experimental.pallas import tpu_sc as plsc

N = 50000
C = 16
E = 1600000
N_TRAIN = 25000
A1 = 0.9
A2 = 0.8
NPROP1 = 10
NPROP2 = 10

N_PAD = 51200
RPT = N_PAD // 16
CHUNK = 784
NBUF = 4


def _cdiv(a, b):
    return (a + b - 1) // b


@functools.lru_cache(maxsize=None)
def _make_agg(cpt: int, gather: bool, sections=None):
    mesh = plsc.VectorSubcoreMesh(
        core_axis_name="c", subcore_axis_name="s", num_cores=2, num_subcores=16
    )

    @functools.partial(
        pl.kernel,
        out_type=jax.ShapeDtypeStruct((2, N_PAD, C), jnp.float32),
        mesh=mesh,
        scratch_types=[
            pltpu.VMEM((NBUF, 2, CHUNK), jnp.int32),
            pltpu.VMEM((NBUF, CHUNK, C), jnp.float32),
            pltpu.VMEM((RPT // 8, C), jnp.float32),
            pltpu.VMEM((C,), jnp.float32),
            pltpu.VMEM_SHARED((N_PAD, C), jnp.float32),
        ] + [pltpu.SemaphoreType.DMA] * (2 * NBUF),
        compiler_params=pltpu.CompilerParams(use_tc_tiling_on_sc=False),
    )
    def agg(g_hbm, sd_hbm, out_hbm, sdv, rows, zbuf, patv, acc, *sems):
        gsem = sems[:NBUF]
        ssem = sems[NBUF:]
        cid = lax.axis_index("c")
        sid = lax.axis_index("s")

        zrows = RPT // 8

        def _zb(i, carry):
            zbuf[i, :] = jnp.zeros((C,), jnp.float32)
            return carry

        lax.fori_loop(0, zrows, _zb, 0)

        for k in range(8):
            pltpu.sync_copy(zbuf, acc.at[pl.ds(sid * RPT + k * zrows, zrows)])
        plsc.subcore_barrier()

        ch0 = cid * (cpt * 16) + sid * cpt

        if not gather:
            def _chunk(i, carry):
                pltpu.sync_copy(sd_hbm.at[ch0 + i], sdv.at[0])
                pltpu.sync_copy(rows.at[0], acc.at[sdv.at[0, 1]], add=True)
                return carry

            for (s0, s1, lane) in sections:
                pltpu.sync_copy(g_hbm.at[lane], patv)

                def _fill(i, carry):
                    rows[0, i, :] = patv[...]
                    return carry

                lax.fori_loop(0, CHUNK, _fill, 0)

                def _chunk_off(i, carry, _s0=s0):
                    return _chunk(i + _s0, carry)

                lax.fori_loop(0, s1 - s0, _chunk_off, 0)
        else:
            def load_fire(ch, b):
                pltpu.sync_copy(sd_hbm.at[ch], sdv.at[b])
                pltpu.async_copy(
                    g_hbm.at[sdv.at[b, 0]], rows.at[b], gsem[b]
                )

            def wait_g(b):
                pltpu.make_async_copy(
                    g_hbm.at[sdv.at[b, 0]], rows.at[b], gsem[b]
                ).wait()

            def fire_s(b):
                pltpu.async_copy(
                    rows.at[b], acc.at[sdv.at[b, 1]], ssem[b], add=True
                )

            def wait_s(b):
                pltpu.make_async_copy(
                    rows.at[b], acc.at[sdv.at[b, 1]], ssem[b]
                ).wait()

            assert cpt >= NBUF and (cpt - NBUF) % NBUF == 0
            load_fire(ch0 + 0, 0)
            load_fire(ch0 + 1, 1)
            wait_g(0)
            fire_s(0)
            load_fire(ch0 + 2, 2)
            wait_g(1)
            fire_s(1)
            load_fire(ch0 + 3, 3)

            def _quad(k, carry):
                ch = ch0 + NBUF + NBUF * k
                for j in range(NBUF):
                    wait_g((j + 2) % NBUF)
                    fire_s((j + 2) % NBUF)
                    wait_s(j)
                    load_fire(ch + j, j)
                return carry

            lax.fori_loop(0, (cpt - NBUF) // NBUF, _quad, 0)
            wait_g(2)
            fire_s(2)
            wait_g(3)
            fire_s(3)
            for j in range(NBUF):
                wait_s(j)
        plsc.subcore_barrier()
        pltpu.sync_copy(
            acc.at[pl.ds(sid * RPT, RPT)],
            out_hbm.at[cid, pl.ds(sid * RPT, RPT)],
        )

    return agg


def _mk_sd(src, dst, total_chunks):
    pad = total_chunks * CHUNK - src.shape[0]
    s = jnp.concatenate([src, jnp.full((pad,), N, jnp.int32)])
    d = jnp.concatenate([dst, jnp.full((pad,), N, jnp.int32)])
    return jnp.stack(
        [s.reshape(total_chunks, CHUNK), d.reshape(total_chunks, CHUNK)],
        axis=1,
    )


_BLK = pl.BlockSpec((RPT, C), lambda i: (i, 0))
_BLK1 = pl.BlockSpec((RPT, 1), lambda i: (i, 0))
_BLKS = pl.BlockSpec((1, 1), lambda i: (0, 0))
_F32 = jnp.float32


def _prep_body(mo, y2, q0, q1,
               err0_o, g0_o, ds_o, dd_o, probs_o, cntm_o, sig_o):
    x = mo[...]
    m = jnp.max(x, axis=1, keepdims=True)
    lse = jnp.log(jnp.sum(jnp.exp(x - m), axis=1, keepdims=True)) + m
    probs = jnp.exp(x - lse)
    yoh = (jax.lax.broadcasted_iota(jnp.int32, x.shape, 1) == y2[...]).astype(_F32)
    q = q0[...] + q1[...]
    degd = q[:, 0:1]
    degs = q[:, 1:2]
    cnt = jnp.broadcast_to(q[:, 2:3], x.shape)
    mask = (cnt > 0.0).astype(_F32)
    err0 = mask * (yoh - probs)
    dsm = jnp.broadcast_to(
        jnp.where(degs > 0.0, jax.lax.rsqrt(jnp.maximum(degs, 1e-30)), 0.0),
        x.shape,
    )
    ddm = jnp.broadcast_to(
        jnp.where(degd > 0.0, jax.lax.rsqrt(jnp.maximum(degd, 1e-30)), 0.0),
        x.shape,
    )
    err0_o[...] = err0
    g0_o[...] = dsm * err0
    ds_o[...] = dsm
    dd_o[...] = ddm
    probs_o[...] = probs
    cntm_o[...] = cnt

    @pl.when(pl.program_id(0) == 0)
    def _():
        sig_o[...] = jnp.zeros((1, 1), _F32)

    row = (pl.program_id(0) * RPT
           + jax.lax.broadcasted_iota(jnp.int32, x.shape, 0))
    contrib = jnp.sum(jnp.where(row < N, cnt * jnp.abs(err0), 0.0))
    sig_o[...] += jnp.full((1, 1), 1.0, _F32) * contrib


def _prep(mo, y2, cntp):
    shp = jax.ShapeDtypeStruct((N_PAD, C), _F32)
    return pl.pallas_call(
        _prep_body,
        grid=(16,),
        in_specs=[_BLK, _BLK1, _BLK, _BLK],
        out_specs=[_BLK, _BLK, _BLK, _BLK, _BLK, _BLK, _BLKS],
        out_shape=[shp, shp, shp, shp, shp, shp,
                   jax.ShapeDtypeStruct((1, 1), _F32)],
    )(mo, y2, cntp[0], cntp[1])


def _step_body(alpha, clip, p0, p1, h0, ddm, dsm, h_o, g_o):
    v = alpha * ddm[...] * (p0[...] + p1[...]) + (1.0 - alpha) * h0[...]
    if clip:
        v = jnp.clip(v, 0.0, 1.0)
    h_o[...] = v
    g_o[...] = dsm[...] * v


def _step(alpha, clip, p, h0, ddm, dsm):
    shp = jax.ShapeDtypeStruct((N_PAD, C), _F32)
    return pl.pallas_call(
        functools.partial(_step_body, alpha, clip),
        grid=(16,),
        in_specs=[_BLK, _BLK, _BLK, _BLK, _BLK],
        out_specs=[_BLK, _BLK],
        out_shape=[shp, shp],
    )(p[0], p[1], h0, ddm, dsm)


def _mid_body(err, probs, cntm, y2, dsm, sig, h0_o, g_o):
    sigma = sig[0, 0] / float(N_TRAIN)
    e = err[...]
    l1 = jnp.sum(jnp.abs(e), axis=1, keepdims=True)
    scale = jnp.clip(sigma / (l1 + 1e-9), 0.0, 1000.0)
    out = probs[...] + scale * e
    yoh = (jax.lax.broadcasted_iota(jnp.int32, e.shape, 1) == y2[...]).astype(_F32)
    h0 = jnp.where(cntm[...] > 0.0, yoh, out)
    h0_o[...] = h0
    g_o[...] = dsm[...] * h0


def _mid(err, probs, cntm, y2, dsm, sig):
    shp = jax.ShapeDtypeStruct((N_PAD, C), _F32)
    return pl.pallas_call(
        _mid_body,
        grid=(16,),
        in_specs=[_BLK, _BLK, _BLK, _BLK1, _BLK, _BLKS],
        out_specs=[_BLK, _BLK],
        out_shape=[shp, shp],
    )(err, probs, cntm, y2, dsm, sig)


def kernel(model_out, y, edge_index, train_idx):
    src = edge_index[0].astype(jnp.int32)
    dst = edge_index[1].astype(jnp.int32)
    train_idx = train_idx.astype(jnp.int32)

    cpt_e = _cdiv(E, 32 * CHUNK)
    cpt_t = _cdiv(N_TRAIN, 32 * CHUNK)

    sd_edges = _mk_sd(src, dst, cpt_e * 32)
    sd_dst = _mk_sd(dst, dst, cpt_e * 32).reshape(32, cpt_e, 2, CHUNK)
    sd_src = _mk_sd(src, src, cpt_e * 32).reshape(32, cpt_e, 2, CHUNK)
    sd_tr = _mk_sd(train_idx, train_idx, cpt_t * 32).reshape(32, cpt_t, 2, CHUNK)
    cpt_q = 2 * cpt_e + cpt_t
    sd_cnt = jnp.concatenate([sd_dst, sd_src, sd_tr], axis=1).reshape(
        32 * cpt_q, 2, CHUNK
    )

    mo = jnp.pad(model_out, ((0, N_PAD - N), (0, 0)))
    y2 = jnp.pad(y, (0, N_PAD - N)).reshape(N_PAD, 1)

    agg = _make_agg(cpt_e, True)
    counters = _make_agg(
        cpt_q, False,
        sections=((0, cpt_e, 0), (cpt_e, 2 * cpt_e, 1), (2 * cpt_e, cpt_q, 2)),
    )

    pat = jnp.eye(4, C, dtype=jnp.float32)
    cntp = counters(pat, sd_cnt)

    err0, g, dsm, ddm, probs, cntm, sig = _prep(mo, y2, cntp)

    err = err0
    for _ in range(NPROP1):
        p = agg(g, sd_edges)
        err, g = _step(A1, False, p, err0, ddm, dsm)

    h0, g = _mid(err, probs, cntm, y2, dsm, sig)

    h = h0
    for _ in range(NPROP2):
        p = agg(g, sd_edges)
        h, g = _step(A2, True, p, h0, ddm, dsm)

    return h[:N]


# speedup vs baseline: 26.7926x; 1.0043x over previous
import functools

import jax
import jax.numpy as jnp
from jax import lax
from jax.experimental import pallas as pl
from jax.experimental.pallas import tpu as pltpu
from jax.experimental.pallas import tpu_sc as plsc

N = 50000
C = 16
E = 1600000
N_TRAIN = 25000
A1 = 0.9
A2 = 0.8
NPROP1 = 10
NPROP2 = 10

N_PAD = 51200
RPT = N_PAD // 16
CHUNK = 784
NBUF = 4


def _cdiv(a, b):
    return (a + b - 1) // b


@functools.lru_cache(maxsize=None)
def _make_agg(cpt: int, gather: bool, sections=None):
    mesh = plsc.VectorSubcoreMesh(
        core_axis_name="c", subcore_axis_name="s", num_cores=2, num_subcores=16
    )

    @functools.partial(
        pl.kernel,
        out_type=jax.ShapeDtypeStruct((2, N_PAD, C), jnp.float32),
        mesh=mesh,
        scratch_types=[
            pltpu.VMEM((NBUF, 2, CHUNK), jnp.int32),
            pltpu.VMEM((NBUF, CHUNK, C), jnp.float32),
            pltpu.VMEM((RPT // 8, C), jnp.float32),
            pltpu.VMEM((C,), jnp.float32),
            pltpu.VMEM_SHARED((N_PAD, C), jnp.float32),
        ] + [pltpu.SemaphoreType.DMA] * (2 * NBUF),
        compiler_params=pltpu.CompilerParams(use_tc_tiling_on_sc=False),
    )
    def agg(g_hbm, sd_hbm, out_hbm, sdv, rows, zbuf, patv, acc, *sems):
        gsem = sems[:NBUF]
        ssem = sems[NBUF:]
        cid = lax.axis_index("c")
        sid = lax.axis_index("s")

        zrows = RPT // 8

        def _zb(i, carry):
            zbuf[i, :] = jnp.zeros((C,), jnp.float32)
            return carry

        lax.fori_loop(0, zrows, _zb, 0)

        for k in range(8):
            pltpu.sync_copy(zbuf, acc.at[pl.ds(sid * RPT + k * zrows, zrows)])
        plsc.subcore_barrier()

        ch0 = cid * (cpt * 16) + sid * cpt

        if not gather:
            def load_sd(ch, b):
                pltpu.sync_copy(sd_hbm.at[ch0 + ch], sdv.at[b])

            def fire_o(b):
                pltpu.async_copy(
                    rows.at[0], acc.at[sdv.at[b, 1]], ssem[b], add=True
                )

            def wait_o(b):
                pltpu.make_async_copy(
                    rows.at[0], acc.at[sdv.at[b, 1]], ssem[b]
                ).wait()

            for (s0, s1, lane) in sections:
                pltpu.sync_copy(g_hbm.at[lane], patv)

                def _fill(i, carry):
                    rows[0, i, :] = patv[...]
                    return carry

                lax.fori_loop(0, CHUNK, _fill, 0)

                n = s1 - s0
                if n >= NBUF and n % NBUF == 0:
                    for j in range(NBUF):
                        load_sd(s0 + j, j)
                        fire_o(j)

                    def _q(k, carry, _s0=s0):
                        ch = _s0 + NBUF + NBUF * k
                        for j in range(NBUF):
                            wait_o(j)
                            load_sd(ch + j, j)
                            fire_o(j)
                        return carry

                    lax.fori_loop(0, n // NBUF - 1, _q, 0)
                    for j in range(NBUF):
                        wait_o(j)
                else:
                    def _chunk_off(i, carry, _s0=s0):
                        load_sd(i + _s0, 0)
                        pltpu.sync_copy(
                            rows.at[0], acc.at[sdv.at[0, 1]], add=True
                        )
                        return carry

                    lax.fori_loop(0, n, _chunk_off, 0)
        else:
            def load_fire(ch, b):
                pltpu.sync_copy(sd_hbm.at[ch], sdv.at[b])
                pltpu.async_copy(
                    g_hbm.at[sdv.at[b, 0]], rows.at[b], gsem[b]
                )

            def wait_g(b):
                pltpu.make_async_copy(
                    g_hbm.at[sdv.at[b, 0]], rows.at[b], gsem[b]
                ).wait()

            def fire_s(b):
                pltpu.async_copy(
                    rows.at[b], acc.at[sdv.at[b, 1]], ssem[b], add=True
                )

            def wait_s(b):
                pltpu.make_async_copy(
                    rows.at[b], acc.at[sdv.at[b, 1]], ssem[b]
                ).wait()

            assert cpt >= NBUF and (cpt - NBUF) % NBUF == 0
            load_fire(ch0 + 0, 0)
            load_fire(ch0 + 1, 1)
            wait_g(0)
            fire_s(0)
            load_fire(ch0 + 2, 2)
            wait_g(1)
            fire_s(1)
            load_fire(ch0 + 3, 3)

            def _quad(k, carry):
                ch = ch0 + NBUF + NBUF * k
                for j in range(NBUF):
                    wait_g((j + 2) % NBUF)
                    fire_s((j + 2) % NBUF)
                    wait_s(j)
                    load_fire(ch + j, j)
                return carry

            lax.fori_loop(0, (cpt - NBUF) // NBUF, _quad, 0)
            wait_g(2)
            fire_s(2)
            wait_g(3)
            fire_s(3)
            for j in range(NBUF):
                wait_s(j)
        plsc.subcore_barrier()
        pltpu.sync_copy(
            acc.at[pl.ds(sid * RPT, RPT)],
            out_hbm.at[cid, pl.ds(sid * RPT, RPT)],
        )

    return agg


def _mk_sd(src, dst, total_chunks):
    pad = total_chunks * CHUNK - src.shape[0]
    s = jnp.concatenate([src, jnp.full((pad,), N, jnp.int32)])
    d = jnp.concatenate([dst, jnp.full((pad,), N, jnp.int32)])
    return jnp.stack(
        [s.reshape(total_chunks, CHUNK), d.reshape(total_chunks, CHUNK)],
        axis=1,
    )


_BLK = pl.BlockSpec((RPT, C), lambda i: (i, 0))
_BLK1 = pl.BlockSpec((RPT, 1), lambda i: (i, 0))
_BLKS = pl.BlockSpec((1, 1), lambda i: (0, 0))
_F32 = jnp.float32


def _prep_body(mo, y2, q0, q1,
               err0_o, g0_o, ds_o, dd_o, probs_o, cntm_o, sig_o):
    x = mo[...]
    m = jnp.max(x, axis=1, keepdims=True)
    lse = jnp.log(jnp.sum(jnp.exp(x - m), axis=1, keepdims=True)) + m
    probs = jnp.exp(x - lse)
    yoh = (jax.lax.broadcasted_iota(jnp.int32, x.shape, 1) == y2[...]).astype(_F32)
    q = q0[...] + q1[...]
    degd = q[:, 0:1]
    degs = q[:, 1:2]
    cnt = jnp.broadcast_to(q[:, 2:3], x.shape)
    mask = (cnt > 0.0).astype(_F32)
    err0 = mask * (yoh - probs)
    dsm = jnp.broadcast_to(
        jnp.where(degs > 0.0, jax.lax.rsqrt(jnp.maximum(degs, 1e-30)), 0.0),
        x.shape,
    )
    ddm = jnp.broadcast_to(
        jnp.where(degd > 0.0, jax.lax.rsqrt(jnp.maximum(degd, 1e-30)), 0.0),
        x.shape,
    )
    err0_o[...] = err0
    g0_o[...] = dsm * err0
    ds_o[...] = dsm
    dd_o[...] = ddm
    probs_o[...] = probs
    cntm_o[...] = cnt

    @pl.when(pl.program_id(0) == 0)
    def _():
        sig_o[...] = jnp.zeros((1, 1), _F32)

    row = (pl.program_id(0) * RPT
           + jax.lax.broadcasted_iota(jnp.int32, x.shape, 0))
    contrib = jnp.sum(jnp.where(row < N, cnt * jnp.abs(err0), 0.0))
    sig_o[...] += jnp.full((1, 1), 1.0, _F32) * contrib


def _prep(mo, y2, cntp):
    shp = jax.ShapeDtypeStruct((N_PAD, C), _F32)
    return pl.pallas_call(
        _prep_body,
        grid=(16,),
        in_specs=[_BLK, _BLK1, _BLK, _BLK],
        out_specs=[_BLK, _BLK, _BLK, _BLK, _BLK, _BLK, _BLKS],
        out_shape=[shp, shp, shp, shp, shp, shp,
                   jax.ShapeDtypeStruct((1, 1), _F32)],
    )(mo, y2, cntp[0], cntp[1])


def _step_body(alpha, clip, p0, p1, h0, ddm, dsm, h_o, g_o):
    v = alpha * ddm[...] * (p0[...] + p1[...]) + (1.0 - alpha) * h0[...]
    if clip:
        v = jnp.clip(v, 0.0, 1.0)
    h_o[...] = v
    g_o[...] = dsm[...] * v


def _step(alpha, clip, p, h0, ddm, dsm):
    shp = jax.ShapeDtypeStruct((N_PAD, C), _F32)
    return pl.pallas_call(
        functools.partial(_step_body, alpha, clip),
        grid=(16,),
        in_specs=[_BLK, _BLK, _BLK, _BLK, _BLK],
        out_specs=[_BLK, _BLK],
        out_shape=[shp, shp],
    )(p[0], p[1], h0, ddm, dsm)


def _mid_body(err, probs, cntm, y2, dsm, sig, h0_o, g_o):
    sigma = sig[0, 0] / float(N_TRAIN)
    e = err[...]
    l1 = jnp.sum(jnp.abs(e), axis=1, keepdims=True)
    scale = jnp.clip(sigma / (l1 + 1e-9), 0.0, 1000.0)
    out = probs[...] + scale * e
    yoh = (jax.lax.broadcasted_iota(jnp.int32, e.shape, 1) == y2[...]).astype(_F32)
    h0 = jnp.where(cntm[...] > 0.0, yoh, out)
    h0_o[...] = h0
    g_o[...] = dsm[...] * h0


def _mid(err, probs, cntm, y2, dsm, sig):
    shp = jax.ShapeDtypeStruct((N_PAD, C), _F32)
    return pl.pallas_call(
        _mid_body,
        grid=(16,),
        in_specs=[_BLK, _BLK, _BLK, _BLK1, _BLK, _BLKS],
        out_specs=[_BLK, _BLK],
        out_shape=[shp, shp],
    )(err, probs, cntm, y2, dsm, sig)


def kernel(model_out, y, edge_index, train_idx):
    src = edge_index[0].astype(jnp.int32)
    dst = edge_index[1].astype(jnp.int32)
    train_idx = train_idx.astype(jnp.int32)

    cpt_e = _cdiv(E, 32 * CHUNK)
    cpt_t = _cdiv(N_TRAIN, 32 * CHUNK)

    sd_edges = _mk_sd(src, dst, cpt_e * 32)
    sd_dst = _mk_sd(dst, dst, cpt_e * 32).reshape(32, cpt_e, 2, CHUNK)
    sd_src = _mk_sd(src, src, cpt_e * 32).reshape(32, cpt_e, 2, CHUNK)
    sd_tr = _mk_sd(train_idx, train_idx, cpt_t * 32).reshape(32, cpt_t, 2, CHUNK)
    cpt_q = 2 * cpt_e + cpt_t
    sd_cnt = jnp.concatenate([sd_dst, sd_src, sd_tr], axis=1).reshape(
        32 * cpt_q, 2, CHUNK
    )

    mo = jnp.pad(model_out, ((0, N_PAD - N), (0, 0)))
    y2 = jnp.pad(y, (0, N_PAD - N)).reshape(N_PAD, 1)

    agg = _make_agg(cpt_e, True)
    counters = _make_agg(
        cpt_q, False,
        sections=((0, cpt_e, 0), (cpt_e, 2 * cpt_e, 1), (2 * cpt_e, cpt_q, 2)),
    )

    pat = jnp.eye(4, C, dtype=jnp.float32)
    cntp = counters(pat, sd_cnt)

    err0, g, dsm, ddm, probs, cntm, sig = _prep(mo, y2, cntp)

    err = err0
    for _ in range(NPROP1):
        p = agg(g, sd_edges)
        err, g = _step(A1, False, p, err0, ddm, dsm)

    h0, g = _mid(err, probs, cntm, y2, dsm, sig)

    h = h0
    for _ in range(NPROP2):
        p = agg(g, sd_edges)
        h, g = _step(A2, True, p, h0, ddm, dsm)

    return h[:N]
